# baseline (speedup 1.0000x reference)
import functools

import jax
import jax.numpy as jnp
from jax import lax
from jax.experimental import pallas as pl
from jax.experimental.pallas import tpu as pltpu
from jax.experimental.pallas import tpu_sc as plsc

_NC = 2
_NS = 16
_NW = _NC * _NS
_CHUNK = 32
_NBUF = 3


@functools.partial(jax.jit, static_argnames=("d_model",))
def _sc_embed(idx, W_E, d_model):
    n_per = idx.shape[1]
    total = _NW * n_per
    sizes = [_CHUNK] * (n_per // _CHUNK)
    if n_per % _CHUNK:
        sizes.append(n_per % _CHUNK)
    offs = [sum(sizes[:j]) for j in range(len(sizes))]
    n = len(sizes)
    mesh = plsc.VectorSubcoreMesh(core_axis_name="c", subcore_axis_name="s")

    @functools.partial(
        pl.kernel,
        out_type=jax.ShapeDtypeStruct((total, d_model), jnp.float32),
        mesh=mesh,
        scratch_types=[
            pltpu.VMEM((n_per,), jnp.int32),
            pltpu.VMEM((_NBUF, _CHUNK, d_model), jnp.float32),
            [pltpu.SemaphoreType.DMA] * _NBUF,
            [pltpu.SemaphoreType.DMA] * _NBUF,
        ],
    )
    def k(idx_hbm, table_hbm, out_hbm, idx_v, bufs, gsems, ssems):
        wid = lax.axis_index("s") * _NC + lax.axis_index("c")
        base = wid * n_per
        pltpu.sync_copy(idx_hbm.at[wid], idx_v)

        def gather(j):
            b = j % _NBUF
            return pltpu.make_async_copy(
                table_hbm.at[idx_v.at[pl.ds(offs[j], sizes[j])]],
                bufs.at[b].at[pl.ds(0, sizes[j])],
                gsems[b],
            )

        def store(j):
            b = j % _NBUF
            return pltpu.make_async_copy(
                bufs.at[b].at[pl.ds(0, sizes[j])],
                out_hbm.at[pl.ds(base + offs[j], sizes[j])],
                ssems[b],
            )

        gather(0).start()
        for j in range(1, n + 1):
            if j < n:
                if j >= _NBUF:
                    store(j - _NBUF).wait()
                gather(j).start()
            gather(j - 1).wait()
            store(j - 1).start()
        for j in range(max(0, n - _NBUF), n):
            store(j).wait()

    return k(idx, W_E)


def kernel(tokens, W_E):
    B, T = tokens.shape
    V, D = W_E.shape
    idx = tokens.reshape(_NW, (B * T) // _NW).astype(jnp.int32)
    out = _sc_embed(idx, W_E, D)
    return out.reshape(B, T, D)


# speedup vs baseline: 1.0168x; 1.0168x over previous
import functools

import jax
import jax.numpy as jnp
from jax import lax
from jax.experimental import pallas as pl
from jax.experimental.pallas import tpu as pltpu
from jax.experimental.pallas import tpu_sc as plsc

_NC = 2
_NS = 16
_NW = _NC * _NS


@functools.partial(jax.jit, static_argnames=("d_model", "chunk"))
def _sc_embed(idx, W_E, d_model, chunk):
    n_chunks = idx.shape[1]
    total = _NW * n_chunks * chunk
    mesh = plsc.VectorSubcoreMesh(core_axis_name="c", subcore_axis_name="s")

    @functools.partial(
        pl.kernel,
        out_type=jax.ShapeDtypeStruct((total, d_model), jnp.float32),
        mesh=mesh,
        scratch_types=[
            pltpu.VMEM((n_chunks, chunk), jnp.int32),
            pltpu.VMEM((chunk, d_model), jnp.float32),
            pltpu.VMEM((chunk, d_model), jnp.float32),
            pltpu.SemaphoreType.DMA,
            pltpu.SemaphoreType.DMA,
        ],
    )
    def k(idx_hbm, table_hbm, out_hbm, idx_v, buf0, buf1, sem0, sem1):
        wid = lax.axis_index("s") * _NC + lax.axis_index("c")
        base = wid * n_chunks * chunk
        pltpu.sync_copy(idx_hbm.at[wid], idx_v)

        def gather(g, buf, sem):
            return pltpu.make_async_copy(table_hbm.at[idx_v.at[g]], buf, sem)

        gather(0, buf0, sem0).start()

        def body(i, carry):
            g = i * 2
            gather(g + 1, buf1, sem1).start()
            gather(g, buf0, sem0).wait()
            pltpu.sync_copy(buf0, out_hbm.at[pl.ds(base + g * chunk, chunk)])

            @pl.when(g + 2 < n_chunks)
            def _():
                gather(g + 2, buf0, sem0).start()

            gather(g + 1, buf1, sem1).wait()
            pltpu.sync_copy(
                buf1, out_hbm.at[pl.ds(base + (g + 1) * chunk, chunk)]
            )
            return carry

        lax.fori_loop(0, n_chunks // 2, body, 0, unroll=False)

    return k(idx, W_E)


def kernel(tokens, W_E):
    B, T = tokens.shape
    V, D = W_E.shape
    total = B * T
    chunk = 32
    n_chunks = total // (_NW * chunk)
    idx = tokens.reshape(_NW, n_chunks, chunk).astype(jnp.int32)
    out = _sc_embed(idx, W_E, D, chunk)
    return out.reshape(B, T, D)


# speedup vs baseline: 1.0274x; 1.0104x over previous
import functools

import jax
import jax.numpy as jnp
from jax import lax
from jax.experimental import pallas as pl
from jax.experimental.pallas import tpu as pltpu
from jax.experimental.pallas import tpu_sc as plsc

_NC = 2
_NS = 16
_NW = _NC * _NS
_CHUNK = 32


@jax.jit
def _sc_embed(tokens, W_E):
    B, T = tokens.shape
    _, D = W_E.shape
    n_per = (B * T) // _NW
    per_row = T // n_per
    n_chunks = n_per // _CHUNK
    mesh = plsc.VectorSubcoreMesh(core_axis_name="c", subcore_axis_name="s")

    @functools.partial(
        pl.kernel,
        out_type=jax.ShapeDtypeStruct((B, T, D), jnp.float32),
        mesh=mesh,
        scratch_types=[
            pltpu.VMEM((n_per,), jnp.int32),
            pltpu.VMEM((_CHUNK, D), jnp.float32),
            pltpu.VMEM((_CHUNK, D), jnp.float32),
            pltpu.SemaphoreType.DMA,
            pltpu.SemaphoreType.DMA,
        ],
    )
    def k(idx_hbm, table_hbm, out_hbm, idx_v, buf0, buf1, sem0, sem1):
        wid = lax.axis_index("s") * _NC + lax.axis_index("c")
        r = wid // per_row
        col = (wid % per_row) * n_per
        pltpu.sync_copy(idx_hbm.at[r].at[pl.ds(col, n_per)], idx_v)

        def gather(g, buf, sem):
            return pltpu.make_async_copy(
                table_hbm.at[idx_v.at[pl.ds(g * _CHUNK, _CHUNK)]], buf, sem
            )

        def out_slice(g):
            return out_hbm.at[r].at[pl.ds(col + g * _CHUNK, _CHUNK)]

        gather(0, buf0, sem0).start()

        def body(i, carry):
            g = i * 2
            gather(g + 1, buf1, sem1).start()
            gather(g, buf0, sem0).wait()
            pltpu.sync_copy(buf0, out_slice(g))

            @pl.when(g + 2 < n_chunks)
            def _():
                gather(g + 2, buf0, sem0).start()

            gather(g + 1, buf1, sem1).wait()
            pltpu.sync_copy(buf1, out_slice(g + 1))
            return carry

        lax.fori_loop(0, n_chunks // 2, body, 0, unroll=False)

    return k(tokens, W_E)


def kernel(tokens, W_E):
    return _sc_embed(tokens.astype(jnp.int32), W_E)
